# baseline (speedup 1.0000x reference)
import functools

import jax
import jax.numpy as jnp
from jax import lax
from jax.experimental import pallas as pl

B, C, H, W = 128, 768, 14, 14
HW = H * W
E, TOPK = 16, 2

BB = 16


def _gate_body(x_ref, wc_ref, bc_ref, noise_ref, w_out, idx_out):
    xb = x_ref[...]
    s = jnp.sum(xb, axis=2)
    m = jnp.max(xb, axis=2)
    f = s * (1.0 / HW) + m

    z = lax.dot_general(
        f, wc_ref[...],
        dimension_numbers=(((1,), (0,)), ((), ())),
        preferred_element_type=jnp.float32,
        precision=lax.Precision.HIGHEST,
    ) + bc_ref[...]

    n1 = z[:, :E]
    n2 = z[:, E:]
    n = n1 + noise_ref[...] * jax.nn.softplus(n2)

    iota = lax.broadcasted_iota(jnp.int32, (BB, E), 1)
    v1 = jnp.max(n, axis=1, keepdims=True)
    i1 = jnp.min(jnp.where(n == v1, iota, E), axis=1, keepdims=True)
    masked = jnp.where(iota == i1, -jnp.inf, n)
    v2 = jnp.max(masked, axis=1, keepdims=True)
    i2 = jnp.min(jnp.where(masked == v2, iota, E), axis=1, keepdims=True)

    e2 = jnp.exp(v2 - v1)
    denom = 1.0 + e2
    w_out[...] = jnp.where(
        iota == i1, 1.0 / denom,
        jnp.where(iota == i2, e2 / denom, 0.0))
    idx_out[...] = jnp.concatenate([i1, i2], axis=1)


@jax.jit
def kernel(x, w1_w, w1_b, w2_w, w2_b, noise):
    xr = x.reshape(B, C, HW)
    wc = jnp.concatenate([w1_w, w2_w], axis=0).T
    bc = jnp.concatenate([w1_b, w2_b]).reshape(1, 2 * E)

    grid = (B // BB,)
    w, idx = pl.pallas_call(
        _gate_body,
        grid=grid,
        in_specs=[
            pl.BlockSpec((BB, C, HW), lambda i: (i, 0, 0)),
            pl.BlockSpec((C, 2 * E), lambda i: (0, 0)),
            pl.BlockSpec((1, 2 * E), lambda i: (0, 0)),
            pl.BlockSpec((BB, E), lambda i: (i, 0)),
        ],
        out_specs=[
            pl.BlockSpec((BB, E), lambda i: (i, 0)),
            pl.BlockSpec((BB, TOPK), lambda i: (i, 0)),
        ],
        out_shape=[
            jax.ShapeDtypeStruct((B, E), jnp.float32),
            jax.ShapeDtypeStruct((B, TOPK), jnp.int32),
        ],
    )(xr, wc, bc, noise)
    return (w, idx)
